# Initial kernel scaffold; baseline (speedup 1.0000x reference)
import jax
import jax.numpy as jnp
from jax.experimental import pallas as pl


def kernel(x, index, weight):
    raise NotImplementedError("write your pallas kernel here")


# speedup vs baseline: 107.7237x; 107.7237x over previous
import functools

import jax
import jax.numpy as jnp
from jax import lax
from jax.experimental import pallas as pl
from jax.experimental.pallas import tpu as pltpu
from jax.experimental.pallas import tpu_sc as plsc

R, C = 16384, 200
N = R * C
NC, NS = 2, 16
NW = NC * NS
PER_W = N // NW
CHUNK = 12800
NCHUNK = PER_W // CHUNK
L = 16


def _sc_body(x_hbm, idx_hbm, w_hbm, out_hbm, idx_v, w_v, x_v, sem):
    wid = lax.axis_index("s") * NC + lax.axis_index("c")
    base = wid * PER_W

    def chunk_body(i, carry):
        off = base + i * CHUNK
        pltpu.sync_copy(idx_hbm.at[pl.ds(off, CHUNK)], idx_v)
        gather = pltpu.async_copy(w_hbm.at[idx_v], w_v, sem)
        pltpu.sync_copy(x_hbm.at[pl.ds(off, CHUNK)], x_v)
        gather.wait()

        def mul_body(j, carry2):
            s = pl.ds(j * L, L)
            w_v[s] = w_v[s] * x_v[s]
            return carry2

        lax.fori_loop(0, CHUNK // L, mul_body, 0, unroll=8)
        pltpu.sync_copy(w_v, out_hbm.at[pl.ds(off, CHUNK)])
        return carry

    lax.fori_loop(0, NCHUNK, chunk_body, 0)


@jax.jit
def _run(xf, idxf, weight):
    mesh = plsc.VectorSubcoreMesh(
        core_axis_name="c", subcore_axis_name="s", num_cores=NC, num_subcores=NS
    )
    return pl.kernel(
        _sc_body,
        out_type=jax.ShapeDtypeStruct((N,), jnp.float32),
        mesh=mesh,
        scratch_types=[
            pltpu.VMEM((CHUNK,), jnp.int32),
            pltpu.VMEM((CHUNK,), jnp.float32),
            pltpu.VMEM((CHUNK,), jnp.float32),
            pltpu.SemaphoreType.DMA,
        ],
    )(xf, idxf, weight)


def kernel(x, index, weight):
    xf = x.reshape(N)
    idxf = index.astype(jnp.int32).reshape(N)
    return _run(xf, idxf, weight).reshape(R, C)


# speedup vs baseline: 120.6440x; 1.1199x over previous
import jax
import jax.numpy as jnp
from jax import lax
from jax.experimental import pallas as pl
from jax.experimental.pallas import tpu as pltpu
from jax.experimental.pallas import tpu_sc as plsc

R, C = 16384, 200
N = R * C
NC, NS = 2, 16
NW = NC * NS
PER_W = N // NW
CHUNK = 12800
NCHUNK = PER_W // CHUNK
L = 16


def _mul_chunk(w_ref, x_ref):
    def mul_body(j, carry):
        s = pl.ds(j * L, L)
        w_ref[s] = w_ref[s] * x_ref[s]
        return carry

    lax.fori_loop(0, CHUNK // L, mul_body, 0, unroll=8)


def _sc_body(x_hbm, idx_hbm, w_hbm, out_hbm,
             idx0, idx1, w0, w1, x0, x1,
             si0, si1, sg0, sg1, sx0, sx1, so0, so1):
    idx_v = (idx0, idx1)
    w_v = (w0, w1)
    x_v = (x0, x1)
    sem_i = (si0, si1)
    sem_g = (sg0, sg1)
    sem_x = (sx0, sx1)
    sem_o = (so0, so1)

    wid = lax.axis_index("s") * NC + lax.axis_index("c")
    base = wid * PER_W

    cp_i, cp_g, cp_x, cp_o = {}, {}, {}, {}

    def start_idx(i, b):
        off = base + i * CHUNK
        cp_i[b] = pltpu.async_copy(idx_hbm.at[pl.ds(off, CHUNK)], idx_v[b], sem_i[b])

    def start_gx(i, b):
        off = base + i * CHUNK
        cp_g[b] = pltpu.async_copy(w_hbm.at[idx_v[b]], w_v[b], sem_g[b])
        cp_x[b] = pltpu.async_copy(x_hbm.at[pl.ds(off, CHUNK)], x_v[b], sem_x[b])

    def mul_out(i, b):
        off = base + i * CHUNK
        _mul_chunk(w_v[b], x_v[b])
        cp_o[b] = pltpu.async_copy(w_v[b], out_hbm.at[pl.ds(off, CHUNK)], sem_o[b])

    start_idx(0, 0)
    cp_i[0].wait()
    start_gx(0, 0)
    start_idx(1, 1)

    for i in range(1, NCHUNK):
        b, pb = i % 2, (i - 1) % 2
        cp_i[b].wait()
        if i >= 2:
            cp_o[b].wait()
        start_gx(i, b)
        cp_g[pb].wait()
        cp_x[pb].wait()
        if i + 1 < NCHUNK:
            start_idx(i + 1, pb)
        mul_out(i - 1, pb)

    b = (NCHUNK - 1) % 2
    cp_g[b].wait()
    cp_x[b].wait()
    mul_out(NCHUNK - 1, b)
    cp_o[1 - b].wait()
    cp_o[b].wait()


@jax.jit
def _run(xf, idxf, weight):
    mesh = plsc.VectorSubcoreMesh(
        core_axis_name="c", subcore_axis_name="s", num_cores=NC, num_subcores=NS
    )
    return pl.kernel(
        _sc_body,
        out_type=jax.ShapeDtypeStruct((N,), jnp.float32),
        mesh=mesh,
        scratch_types=[
            pltpu.VMEM((CHUNK,), jnp.int32),
            pltpu.VMEM((CHUNK,), jnp.int32),
            pltpu.VMEM((CHUNK,), jnp.float32),
            pltpu.VMEM((CHUNK,), jnp.float32),
            pltpu.VMEM((CHUNK,), jnp.float32),
            pltpu.VMEM((CHUNK,), jnp.float32),
        ] + [pltpu.SemaphoreType.DMA] * 8,
    )(xf, idxf, weight)


def kernel(x, index, weight):
    xf = x.reshape(N)
    idxf = index.astype(jnp.int32).reshape(N)
    return _run(xf, idxf, weight).reshape(R, C)


# speedup vs baseline: 176.8886x; 1.4662x over previous
import jax
import jax.numpy as jnp
from jax import lax
from jax.experimental import pallas as pl
from jax.experimental.pallas import tpu as pltpu
from jax.experimental.pallas import tpu_sc as plsc

WEIGHT_LEN = 1_000_000
R, C = 16384, 200
N = R * C
NC, NS = 2, 16
NW = NC * NS
PER_W = N // NW
CHUNK = 10240
NCHUNK = PER_W // CHUNK
L = 16


def _mul_chunk(w_ref, x_ref):
    def mul_body(j, carry):
        s = pl.ds(j * L, L)
        w_ref[s] = w_ref[s] * x_ref[s]
        return carry

    lax.fori_loop(0, CHUNK // L, mul_body, 0, unroll=8)


def _sc_body(x_hbm, idx_hbm, w_hbm, out_hbm,
             table_sh, idx0, idx1, w0, w1, x0, x1,
             si0, si1, sg0, sg1, sx0, sx1, so0, so1):
    idx_v = (idx0, idx1)
    w_v = (w0, w1)
    x_v = (x0, x1)
    sem_i = (si0, si1)
    sem_g = (sg0, sg1)
    sem_x = (sx0, sx1)
    sem_o = (so0, so1)

    wid = lax.axis_index("s") * NC + lax.axis_index("c")
    base = wid * PER_W

    sid = lax.axis_index("s")

    @pl.when(sid < 10)
    def _():
        def stage_piece(p, carry):
            off = sid * 100_000 + p * 10_000
            pltpu.sync_copy(w_hbm.at[pl.ds(off, 10_000)], w0.at[pl.ds(0, 10_000)])
            pltpu.sync_copy(w0.at[pl.ds(0, 10_000)], table_sh.at[pl.ds(off, 10_000)])
            return carry

        lax.fori_loop(0, 10, stage_piece, 0)

    plsc.subcore_barrier()

    cp_i, cp_g, cp_x, cp_o = {}, {}, {}, {}

    def start_idx(i, b):
        off = base + i * CHUNK
        cp_i[b] = pltpu.async_copy(idx_hbm.at[pl.ds(off, CHUNK)], idx_v[b], sem_i[b])

    def start_gx(i, b):
        off = base + i * CHUNK
        cp_g[b] = pltpu.async_copy(table_sh.at[idx_v[b]], w_v[b], sem_g[b])
        cp_x[b] = pltpu.async_copy(x_hbm.at[pl.ds(off, CHUNK)], x_v[b], sem_x[b])

    def mul_out(i, b):
        off = base + i * CHUNK
        _mul_chunk(w_v[b], x_v[b])
        cp_o[b] = pltpu.async_copy(w_v[b], out_hbm.at[pl.ds(off, CHUNK)], sem_o[b])

    start_idx(0, 0)
    cp_i[0].wait()
    start_gx(0, 0)
    start_idx(1, 1)

    for i in range(1, NCHUNK):
        b, pb = i % 2, (i - 1) % 2
        cp_i[b].wait()
        if i >= 2:
            cp_o[b].wait()
        start_gx(i, b)
        cp_g[pb].wait()
        cp_x[pb].wait()
        if i + 1 < NCHUNK:
            start_idx(i + 1, pb)
        mul_out(i - 1, pb)

    b = (NCHUNK - 1) % 2
    cp_g[b].wait()
    cp_x[b].wait()
    mul_out(NCHUNK - 1, b)
    cp_o[1 - b].wait()
    cp_o[b].wait()


@jax.jit
def _run(xf, idxf, weight):
    mesh = plsc.VectorSubcoreMesh(
        core_axis_name="c", subcore_axis_name="s", num_cores=NC, num_subcores=NS
    )
    return pl.kernel(
        _sc_body,
        out_type=jax.ShapeDtypeStruct((N,), jnp.float32),
        mesh=mesh,
        scratch_types=[
            pltpu.VMEM_SHARED((WEIGHT_LEN,), jnp.float32),
            pltpu.VMEM((CHUNK,), jnp.int32),
            pltpu.VMEM((CHUNK,), jnp.int32),
            pltpu.VMEM((CHUNK,), jnp.float32),
            pltpu.VMEM((CHUNK,), jnp.float32),
            pltpu.VMEM((CHUNK,), jnp.float32),
            pltpu.VMEM((CHUNK,), jnp.float32),
        ] + [pltpu.SemaphoreType.DMA] * 8,
    )(xf, idxf, weight)


def kernel(x, index, weight):
    xf = x.reshape(N)
    idxf = index.astype(jnp.int32).reshape(N)
    return _run(xf, idxf, weight).reshape(R, C)


# speedup vs baseline: 232.7742x; 1.3159x over previous
import jax
import jax.numpy as jnp
from jax import lax
from jax.experimental import pallas as pl
from jax.experimental.pallas import tpu as pltpu
from jax.experimental.pallas import tpu_sc as plsc

WEIGHT_LEN = 1_000_000
R, C = 16384, 200
RT = R // 8
CT = 2
NT = RT * CT * 8 * 128
PAD = 128 - (C - 128)

NC, NS = 2, 16
NW = NC * NS
PER_W = NT // NW
CHUNK = 16384
NCHUNK = PER_W // CHUNK

GRID = 8
RB = R // GRID
TB = RT // GRID


def _detile_body(idx_ref, out_ref):
    pid = pl.program_id(0)
    a = idx_ref[:, :128].reshape(TB, 8, 128)
    out_ref[:, 0] = a
    base = pid * RB * PAD
    pad = (base
           + PAD * lax.broadcasted_iota(jnp.int32, (RB, PAD), 0)
           + lax.broadcasted_iota(jnp.int32, (RB, PAD), 1))
    b = jnp.concatenate([idx_ref[:, 128:], pad], axis=1).reshape(TB, 8, 128)
    out_ref[:, 1] = b


def _mul_body(w_ref, x_ref, out_ref):
    a = w_ref[:, 0].reshape(RB, 128)
    b = w_ref[:, 1].reshape(RB, 128)[:, : C - 128]
    out_ref[...] = jnp.concatenate([a, b], axis=1) * x_ref[...]


def _sc_body(idx_hbm, w_hbm, out_hbm,
             table_sh, idx0, idx1, w0, w1,
             si0, si1, sg0, sg1, so0, so1):
    idx_v = (idx0, idx1)
    w_v = (w0, w1)
    sem_i = (si0, si1)
    sem_g = (sg0, sg1)
    sem_o = (so0, so1)

    wid = lax.axis_index("s") * NC + lax.axis_index("c")
    base = wid * PER_W

    sid = lax.axis_index("s")

    @pl.when(sid < 10)
    def _():
        def stage_piece(p, carry):
            off = sid * 100_000 + p * 10_000
            pltpu.sync_copy(w_hbm.at[pl.ds(off, 10_000)], w0.at[pl.ds(0, 10_000)])
            pltpu.sync_copy(w0.at[pl.ds(0, 10_000)], table_sh.at[pl.ds(off, 10_000)])
            return carry

        lax.fori_loop(0, 10, stage_piece, 0)

    plsc.subcore_barrier()

    cp_i, cp_g, cp_o = {}, {}, {}

    def start_idx(i, b):
        off = base + i * CHUNK
        cp_i[b] = pltpu.async_copy(idx_hbm.at[pl.ds(off, CHUNK)], idx_v[b], sem_i[b])

    def start_gather(b):
        cp_g[b] = pltpu.async_copy(table_sh.at[idx_v[b]], w_v[b], sem_g[b])

    def start_out(i, b):
        off = base + i * CHUNK
        cp_o[b] = pltpu.async_copy(w_v[b], out_hbm.at[pl.ds(off, CHUNK)], sem_o[b])

    start_idx(0, 0)
    for i in range(NCHUNK):
        b, pb = i % 2, (i - 1) % 2
        cp_i[b].wait()
        if i >= 2:
            cp_o[b].wait()
        start_gather(b)
        if i >= 1:
            cp_g[pb].wait()
            start_out(i - 1, pb)
        if i + 1 < NCHUNK:
            start_idx(i + 1, pb)
    b = (NCHUNK - 1) % 2
    cp_g[b].wait()
    start_out(NCHUNK - 1, b)
    cp_o[1 - b].wait()
    cp_o[b].wait()


@jax.jit
def _run(x, idx, weight):
    idx_t = pl.pallas_call(
        _detile_body,
        grid=(GRID,),
        in_specs=[pl.BlockSpec((RB, C), lambda i: (i, 0))],
        out_specs=pl.BlockSpec((TB, CT, 8, 128), lambda i: (i, 0, 0, 0)),
        out_shape=jax.ShapeDtypeStruct((RT, CT, 8, 128), jnp.int32),
    )(idx)

    mesh = plsc.VectorSubcoreMesh(
        core_axis_name="c", subcore_axis_name="s", num_cores=NC, num_subcores=NS
    )
    w_t = pl.kernel(
        _sc_body,
        out_type=jax.ShapeDtypeStruct((NT,), jnp.float32),
        mesh=mesh,
        scratch_types=[
            pltpu.VMEM_SHARED((WEIGHT_LEN,), jnp.float32),
            pltpu.VMEM((CHUNK,), jnp.int32),
            pltpu.VMEM((CHUNK,), jnp.int32),
            pltpu.VMEM((CHUNK,), jnp.float32),
            pltpu.VMEM((CHUNK,), jnp.float32),
        ] + [pltpu.SemaphoreType.DMA] * 6,
    )(idx_t.reshape(NT), weight)

    return pl.pallas_call(
        _mul_body,
        grid=(GRID,),
        in_specs=[
            pl.BlockSpec((TB, CT, 8, 128), lambda i: (i, 0, 0, 0)),
            pl.BlockSpec((RB, C), lambda i: (i, 0)),
        ],
        out_specs=pl.BlockSpec((RB, C), lambda i: (i, 0)),
        out_shape=jax.ShapeDtypeStruct((R, C), jnp.float32),
    )(w_t.reshape(RT, CT, 8, 128), x)


def kernel(x, index, weight):
    return _run(x, index.astype(jnp.int32), weight)


# speedup vs baseline: 368.6289x; 1.5836x over previous
import jax
import jax.numpy as jnp
from jax import lax
from jax.experimental import pallas as pl
from jax.experimental.pallas import tpu as pltpu
from jax.experimental.pallas import tpu_sc as plsc

WEIGHT_LEN = 1_000_000
R, C = 16384, 200
N = R * C
ST = C // 8
LT = R // 128

NC, NS = 2, 16
NW = NC * NS
PER_W = N // NW
CHUNK = 12800
NCHUNK = PER_W // CHUNK

GRID = 8
LTB = LT // GRID
XB = LTB * 128


def _detile_body(idx_ref, out_ref):
    a = idx_ref[...].reshape(ST, 8, LTB, 128)
    out_ref[...] = a.transpose(0, 2, 1, 3)


def _mul_body(w_ref, x_ref, out_ref):
    g = w_ref[...].transpose(0, 2, 1, 3).reshape(C, XB)
    out_ref[...] = g * x_ref[...]


def _sc_body(idx_hbm, w_hbm, out_hbm,
             table_sh, idx0, idx1, w0, w1,
             si0, si1, sg0, sg1, so0, so1):
    idx_v = (idx0, idx1)
    w_v = (w0, w1)
    sem_i = (si0, si1)
    sem_g = (sg0, sg1)
    sem_o = (so0, so1)

    wid = lax.axis_index("s") * NC + lax.axis_index("c")
    base = wid * PER_W

    cp_i, cp_g, cp_o = {}, {}, {}

    def start_idx(i, b):
        off = base + i * CHUNK
        cp_i[b] = pltpu.async_copy(idx_hbm.at[pl.ds(off, CHUNK)], idx_v[b], sem_i[b])

    def start_gather(b):
        cp_g[b] = pltpu.async_copy(table_sh.at[idx_v[b]], w_v[b], sem_g[b])

    def start_out(i, b):
        off = base + i * CHUNK
        cp_o[b] = pltpu.async_copy(w_v[b], out_hbm.at[pl.ds(off, CHUNK)], sem_o[b])

    start_idx(0, 0)
    start_idx(1, 1)

    sid = lax.axis_index("s")

    def stage_pair(j2, carry):
        j0, j1 = 2 * j2, 2 * j2 + 1
        off0 = sid * 10_000 + j0 * 160_000
        off1 = sid * 10_000 + j1 * 160_000
        have0 = sid + NS * j0 < 100
        have1 = sid + NS * j1 < 100

        @pl.when(have0)
        def _():
            pltpu.async_copy(w_hbm.at[pl.ds(off0, 10_000)],
                             w0.at[pl.ds(0, 10_000)], sem_g[0])

        @pl.when(have1)
        def _():
            pltpu.async_copy(w_hbm.at[pl.ds(off1, 10_000)],
                             w1.at[pl.ds(0, 10_000)], sem_g[1])

        @pl.when(have0)
        def _():
            pltpu.make_async_copy(w_hbm.at[pl.ds(0, 10_000)],
                                  w0.at[pl.ds(0, 10_000)], sem_g[0]).wait()
            pltpu.async_copy(w0.at[pl.ds(0, 10_000)],
                             table_sh.at[pl.ds(off0, 10_000)], sem_o[0])

        @pl.when(have1)
        def _():
            pltpu.make_async_copy(w_hbm.at[pl.ds(0, 10_000)],
                                  w1.at[pl.ds(0, 10_000)], sem_g[1]).wait()
            pltpu.async_copy(w1.at[pl.ds(0, 10_000)],
                             table_sh.at[pl.ds(off1, 10_000)], sem_o[1])

        @pl.when(have0)
        def _():
            pltpu.make_async_copy(w0.at[pl.ds(0, 10_000)],
                                  table_sh.at[pl.ds(off0, 10_000)], sem_o[0]).wait()

        @pl.when(have1)
        def _():
            pltpu.make_async_copy(w1.at[pl.ds(0, 10_000)],
                                  table_sh.at[pl.ds(off1, 10_000)], sem_o[1]).wait()

        return carry

    lax.fori_loop(0, 4, stage_pair, 0)

    plsc.subcore_barrier()

    for i in range(NCHUNK):
        b, pb = i % 2, (i - 1) % 2
        cp_i[b].wait()
        if i >= 2:
            cp_o[b].wait()
        start_gather(b)
        if i >= 1:
            cp_g[pb].wait()
            start_out(i - 1, pb)
        if 1 <= i and i + 1 < NCHUNK:
            start_idx(i + 1, pb)
    b = (NCHUNK - 1) % 2
    cp_g[b].wait()
    start_out(NCHUNK - 1, b)
    cp_o[1 - b].wait()
    cp_o[b].wait()


@jax.jit
def _run(x, idx, weight):
    xt = x.T
    idxt = idx.T

    idx_t = pl.pallas_call(
        _detile_body,
        grid=(GRID,),
        in_specs=[pl.BlockSpec((C, XB), lambda i: (0, i))],
        out_specs=pl.BlockSpec((ST, LTB, 8, 128), lambda i: (0, i, 0, 0)),
        out_shape=jax.ShapeDtypeStruct((ST, LT, 8, 128), jnp.int32),
    )(idxt)

    mesh = plsc.VectorSubcoreMesh(
        core_axis_name="c", subcore_axis_name="s", num_cores=NC, num_subcores=NS
    )
    w_lin = pl.kernel(
        _sc_body,
        out_type=jax.ShapeDtypeStruct((N,), jnp.float32),
        mesh=mesh,
        scratch_types=[
            pltpu.VMEM_SHARED((WEIGHT_LEN,), jnp.float32),
            pltpu.VMEM((CHUNK,), jnp.int32),
            pltpu.VMEM((CHUNK,), jnp.int32),
            pltpu.VMEM((CHUNK,), jnp.float32),
            pltpu.VMEM((CHUNK,), jnp.float32),
        ] + [pltpu.SemaphoreType.DMA] * 6,
    )(idx_t.reshape(N), weight)

    out_t = pl.pallas_call(
        _mul_body,
        grid=(GRID,),
        in_specs=[
            pl.BlockSpec((ST, LTB, 8, 128), lambda i: (0, i, 0, 0)),
            pl.BlockSpec((C, XB), lambda i: (0, i)),
        ],
        out_specs=pl.BlockSpec((C, XB), lambda i: (0, i)),
        out_shape=jax.ShapeDtypeStruct((C, R), jnp.float32),
    )(w_lin.reshape(ST, LT, 8, 128), xt)

    return out_t.T


def kernel(x, index, weight):
    return _run(x, index.astype(jnp.int32), weight)
